# baseline (speedup 1.0000x reference)
import functools

import jax
import jax.numpy as jnp
from jax import lax
from jax.experimental import pallas as pl
from jax.experimental.pallas import tpu as pltpu
from jax.experimental.pallas import tpu_sc as plsc

SEQ_LEN = 2048
D_MODEL = 1024
BATCH = 32

NUM_CORES = 2
NUM_SUBCORES = 16
NUM_WORKERS = NUM_CORES * NUM_SUBCORES
ROWS_PER_W = SEQ_LEN // NUM_WORKERS

_mesh = plsc.VectorSubcoreMesh(
    core_axis_name="c", subcore_axis_name="s",
    num_cores=NUM_CORES, num_subcores=NUM_SUBCORES,
)


@functools.partial(
    pl.kernel,
    out_type=jax.ShapeDtypeStruct((SEQ_LEN, BATCH, D_MODEL), jnp.float32),
    mesh=_mesh,
    scratch_types=[
        pltpu.VMEM((ROWS_PER_W, D_MODEL), jnp.float32),
        pltpu.SemaphoreType.DMA,
    ],
)
def _sc_broadcast(pe_hbm, out_hbm, buf, sem):
    wid = lax.axis_index("c") * NUM_SUBCORES + lax.axis_index("s")
    base = wid * ROWS_PER_W
    pltpu.sync_copy(pe_hbm.at[pl.ds(base, ROWS_PER_W)], buf)
    for n in range(BATCH):
        pltpu.async_copy(buf, out_hbm.at[pl.ds(base, ROWS_PER_W), n], sem)
    for n in range(BATCH):
        pltpu.make_async_copy(buf, out_hbm.at[pl.ds(base, ROWS_PER_W), n], sem).wait()


def kernel(z, pos_embed):
    del z
    return _sc_broadcast(pos_embed)


# speedup vs baseline: 1.0047x; 1.0047x over previous
import functools

import jax
import jax.numpy as jnp
from jax import lax
from jax.experimental import pallas as pl
from jax.experimental.pallas import tpu as pltpu
from jax.experimental.pallas import tpu_sc as plsc

SEQ_LEN = 2048
D_MODEL = 1024
BATCH = 32

NUM_CORES = 2
NUM_SUBCORES = 16
NUM_WORKERS = NUM_CORES * NUM_SUBCORES
ROWS_PER_W = SEQ_LEN // NUM_WORKERS

_mesh = plsc.VectorSubcoreMesh(
    core_axis_name="c", subcore_axis_name="s",
    num_cores=NUM_CORES, num_subcores=NUM_SUBCORES,
)


@functools.partial(
    pl.kernel,
    out_type=jax.ShapeDtypeStruct((SEQ_LEN, BATCH, D_MODEL), jnp.float32),
    mesh=_mesh,
    scratch_types=[
        pltpu.VMEM((ROWS_PER_W, D_MODEL), jnp.float32),
        pltpu.SemaphoreType.DMA,
    ],
    compiler_params=pltpu.CompilerParams(use_tc_tiling_on_sc=True),
)
def _sc_broadcast(pe_hbm, out_hbm, buf, sem):
    wid = lax.axis_index("c") * NUM_SUBCORES + lax.axis_index("s")
    base = wid * ROWS_PER_W
    pltpu.sync_copy(pe_hbm.at[pl.ds(base, ROWS_PER_W)], buf)
    for n in range(BATCH):
        pltpu.async_copy(buf, out_hbm.at[pl.ds(base, ROWS_PER_W), n], sem)
    for n in range(BATCH):
        pltpu.make_async_copy(buf, out_hbm.at[pl.ds(base, ROWS_PER_W), n], sem).wait()


def kernel(z, pos_embed):
    del z
    return _sc_broadcast(pos_embed)
